# Initial kernel scaffold; baseline (speedup 1.0000x reference)
import functools

import jax
import jax.numpy as jnp
from jax import lax
from jax.experimental import pallas as pl
from jax.experimental.pallas import tpu as pltpu

STEPS_ = 5
MT = 512


def _icp_step_body(st_ref, gt_ref, a_ref, b_ref, h_ref, sx_ref, sk_ref):
    st = st_ref[0]
    gt = gt_ref[0]
    A = a_ref[0]
    bv = b_ref[0]

    N = st.shape[1]
    M = gt.shape[1]

    tt = lax.dot_general(A, st, (((1,), (0,)), ((), ())),
                         preferred_element_type=jnp.float32)
    tt = tt + bv.reshape(3, 1)

    t2 = jnp.sum(gt * gt, axis=0)

    run_min = jnp.full((N,), jnp.inf, jnp.float32)
    run_idx = jnp.zeros((N,), jnp.int32)
    for j in range(M // MT):
        gtile = gt[:, j * MT:(j + 1) * MT]
        prod = lax.dot_general(tt, gtile, (((0,), (0,)), ((), ())),
                               preferred_element_type=jnp.float32)
        d2 = t2[j * MT:(j + 1) * MT][None, :] - 2.0 * prod
        tmin = jnp.min(d2, axis=1)
        targ = jnp.argmin(d2, axis=1).astype(jnp.int32) + j * MT
        better = tmin < run_min
        run_min = jnp.where(better, tmin, run_min)
        run_idx = jnp.where(better, targ, run_idx)

    h_acc = jnp.zeros((3, 3), jnp.float32)
    sk_acc = jnp.zeros((3,), jnp.float32)
    iota = lax.broadcasted_iota(jnp.int32, (N, MT), 1)
    for j in range(M // MT):
        gtile = gt[:, j * MT:(j + 1) * MT]
        onehot = (run_idx[:, None] == (iota + j * MT)).astype(jnp.float32)
        m3 = lax.dot_general(tt, onehot, (((1,), (0,)), ((), ())),
                             preferred_element_type=jnp.float32)
        h_acc = h_acc + lax.dot_general(m3, gtile, (((1,), (1,)), ((), ())),
                                        preferred_element_type=jnp.float32)
        colsum = jnp.sum(onehot, axis=0)
        sk_acc = sk_acc + jnp.sum(gtile * colsum[None, :], axis=1)

    h_ref[0] = h_acc
    sx_ref[0, 0] = jnp.sum(tt, axis=1)
    sk_ref[0, 0] = sk_acc


def _icp_step(st, gt, A, bv, interpret=False):
    B, _, N = st.shape
    M = gt.shape[2]
    return pl.pallas_call(
        _icp_step_body,
        grid=(B,),
        in_specs=[
            pl.BlockSpec((1, 3, N), lambda i: (i, 0, 0)),
            pl.BlockSpec((1, 3, M), lambda i: (i, 0, 0)),
            pl.BlockSpec((1, 3, 3), lambda i: (i, 0, 0)),
            pl.BlockSpec((1, 1, 3), lambda i: (i, 0, 0)),
        ],
        out_specs=[
            pl.BlockSpec((1, 3, 3), lambda i: (i, 0, 0)),
            pl.BlockSpec((1, 1, 3), lambda i: (i, 0, 0)),
            pl.BlockSpec((1, 1, 3), lambda i: (i, 0, 0)),
        ],
        out_shape=[
            jax.ShapeDtypeStruct((B, 3, 3), jnp.float32),
            jax.ShapeDtypeStruct((B, 1, 3), jnp.float32),
            jax.ShapeDtypeStruct((B, 1, 3), jnp.float32),
        ],
        compiler_params=pltpu.CompilerParams(
            dimension_semantics=("arbitrary",),
        ),
        interpret=interpret,
    )(st, gt, A, bv)


def _kabsch(Hc, cs, ct):
    U, S, Vt = jnp.linalg.svd(Hc, full_matrices=False)
    V = jnp.swapaxes(Vt, -1, -2)
    Ut = jnp.swapaxes(U, -1, -2)
    det = jnp.linalg.det(jnp.matmul(V, Ut))
    diag = jnp.concatenate(
        [jnp.ones(det.shape + (2,), det.dtype), det[..., None]], axis=-1)
    Rm = jnp.einsum('...ij,...j,...jk->...ik', V, diag, Ut)
    t = ct - jnp.einsum('...ij,...j->...i', Rm, cs)
    return Rm, t


def kernel(source, target, _interpret=False):
    B, N, _ = source.shape
    st = jnp.swapaxes(source, 1, 2)
    gt = jnp.swapaxes(target, 1, 2)
    A = jnp.broadcast_to(jnp.eye(3, dtype=jnp.float32), (B, 3, 3))
    bv = jnp.zeros((B, 1, 3), jnp.float32)
    n_f = jnp.float32(N)
    for _ in range(STEPS_):
        H, Sx, Sk = _icp_step(st, gt, A, bv, interpret=_interpret)
        cs = Sx[:, 0, :] / n_f
        ct = Sk[:, 0, :] / n_f
        Hc = H - n_f * cs[:, :, None] * ct[:, None, :]
        Rm, t = _kabsch(Hc, cs, ct)
        A = jnp.einsum('bij,bjk->bik', Rm, A)
        bv = (jnp.einsum('bij,bj->bi', Rm, bv[:, 0, :]) + t)[:, None, :]
    return jnp.concatenate([A, bv[:, 0, :, None]], axis=-1)


# speedup vs baseline: 2.5237x; 2.5237x over previous
import functools

import jax
import jax.numpy as jnp
from jax import lax
from jax.experimental import pallas as pl
from jax.experimental.pallas import tpu as pltpu

STEPS_ = 5
MT = 512
BF = jnp.bfloat16
F32 = jnp.float32


def _match_body(st_ref, gt_ref, r_ref, t_ref,
                h_ref, cs_ref, ct_ref, tt_ref, d2_ref, apply_tf):
    st = st_ref[0]
    gt = gt_ref[0]
    N = st.shape[1]
    M = gt.shape[1]

    if apply_tf:
        R = r_ref[0]
        tv = t_ref[0]
        tt = lax.dot_general(R.astype(BF), st.astype(BF),
                             (((1,), (0,)), ((), ())),
                             preferred_element_type=F32)
        tt = tt + tv.reshape(3, 1)
    else:
        tt = st

    ttb = tt.astype(BF)
    gtb = gt.astype(BF)
    s2 = tt[0] * tt[0] + tt[1] * tt[1] + tt[2] * tt[2]
    t2 = gt[0] * gt[0] + gt[1] * gt[1] + gt[2] * gt[2]

    run_min = jnp.full((N,), jnp.inf, F32)
    for j in range(M // MT):
        prod = lax.dot_general(ttb, gtb[:, j * MT:(j + 1) * MT],
                               (((0,), (0,)), ((), ())),
                               preferred_element_type=F32)
        d2 = s2[:, None] + t2[j * MT:(j + 1) * MT][None, :] - 2.0 * prod
        d2_ref[:, j * MT:(j + 1) * MT] = d2
        run_min = jnp.minimum(run_min, jnp.min(d2, axis=1))

    cs = jnp.sum(tt, axis=1) / F32(N)
    sc = tt - cs[:, None]
    scb = sc.astype(BF)
    aug = jnp.concatenate([scb, jnp.ones((1, N), BF)], axis=0)

    sb_tiles = []
    sk = jnp.zeros((3,), F32)
    for j in range(M // MT):
        d2 = d2_ref[:, j * MT:(j + 1) * MT]
        ob = (d2 == run_min[:, None]).astype(BF)
        sb4 = lax.dot_general(aug, ob, (((1,), (0,)), ((), ())),
                              preferred_element_type=F32)
        cnt = sb4[3]
        gtile = gt[:, j * MT:(j + 1) * MT]
        sk = sk + jnp.sum(gtile * cnt[None, :], axis=1)
        sb_tiles.append(sb4[0:3])
    ct = sk / F32(N)

    h_cols = [jnp.zeros((3,), F32) for _ in range(3)]
    for j in range(M // MT):
        gtile = gt[:, j * MT:(j + 1) * MT]
        tcf = (gtile - ct[:, None]).astype(BF).astype(F32)
        sb = sb_tiles[j]
        for c in range(3):
            h_cols[c] = h_cols[c] + jnp.sum(sb * tcf[c:c + 1, :], axis=1)
    H = jnp.concatenate([h_cols[0][:, None], h_cols[1][:, None],
                         h_cols[2][:, None]], axis=1)

    h_ref[0] = H
    cs_ref[0, 0] = cs
    ct_ref[0, 0] = ct
    tt_ref[0] = tt


def _match_call(st, gt, Rp, tp, apply_tf):
    B, _, N = st.shape
    M = gt.shape[2]
    return pl.pallas_call(
        functools.partial(_match_body, apply_tf=apply_tf),
        grid=(B,),
        in_specs=[
            pl.BlockSpec((1, 3, N), lambda i: (i, 0, 0)),
            pl.BlockSpec((1, 3, M), lambda i: (i, 0, 0)),
            pl.BlockSpec((1, 3, 3), lambda i: (i, 0, 0)),
            pl.BlockSpec((1, 1, 3), lambda i: (i, 0, 0)),
        ],
        out_specs=[
            pl.BlockSpec((1, 3, 3), lambda i: (i, 0, 0)),
            pl.BlockSpec((1, 1, 3), lambda i: (i, 0, 0)),
            pl.BlockSpec((1, 1, 3), lambda i: (i, 0, 0)),
            pl.BlockSpec((1, 3, N), lambda i: (i, 0, 0)),
        ],
        out_shape=[
            jax.ShapeDtypeStruct((B, 3, 3), F32),
            jax.ShapeDtypeStruct((B, 1, 3), F32),
            jax.ShapeDtypeStruct((B, 1, 3), F32),
            jax.ShapeDtypeStruct((B, 3, N), F32),
        ],
        scratch_shapes=[pltpu.VMEM((N, M), F32)],
        compiler_params=pltpu.CompilerParams(
            dimension_semantics=("arbitrary",),
        ),
    )(st, gt, Rp, tp)


def _final_body(st0_ref, st_ref, r_ref, t_ref, h_ref, cs_ref, ct_ref):
    st0 = st0_ref[0]
    st = st_ref[0]
    R = r_ref[0]
    tv = t_ref[0]
    N = st.shape[1]

    tt = lax.dot_general(R.astype(BF), st.astype(BF),
                         (((1,), (0,)), ((), ())),
                         preferred_element_type=F32)
    tt = tt + tv.reshape(3, 1)

    cs = jnp.sum(st0, axis=1) / F32(N)
    ct = jnp.sum(tt, axis=1) / F32(N)
    scb = (st0 - cs[:, None]).astype(BF)
    tcb = (tt - ct[:, None]).astype(BF)
    H = lax.dot_general(scb, tcb, (((1,), (1,)), ((), ())),
                        preferred_element_type=F32)
    h_ref[0] = H
    cs_ref[0, 0] = cs
    ct_ref[0, 0] = ct


def _final_call(st0, st, Rp, tp):
    B, _, N = st.shape
    return pl.pallas_call(
        _final_body,
        grid=(B,),
        in_specs=[
            pl.BlockSpec((1, 3, N), lambda i: (i, 0, 0)),
            pl.BlockSpec((1, 3, N), lambda i: (i, 0, 0)),
            pl.BlockSpec((1, 3, 3), lambda i: (i, 0, 0)),
            pl.BlockSpec((1, 1, 3), lambda i: (i, 0, 0)),
        ],
        out_specs=[
            pl.BlockSpec((1, 3, 3), lambda i: (i, 0, 0)),
            pl.BlockSpec((1, 1, 3), lambda i: (i, 0, 0)),
            pl.BlockSpec((1, 1, 3), lambda i: (i, 0, 0)),
        ],
        out_shape=[
            jax.ShapeDtypeStruct((B, 3, 3), F32),
            jax.ShapeDtypeStruct((B, 1, 3), F32),
            jax.ShapeDtypeStruct((B, 1, 3), F32),
        ],
        compiler_params=pltpu.CompilerParams(
            dimension_semantics=("arbitrary",),
        ),
    )(st0, st, Rp, tp)


def _inv_t(X):
    a, b, c = X[..., 0, 0], X[..., 0, 1], X[..., 0, 2]
    d, e, f = X[..., 1, 0], X[..., 1, 1], X[..., 1, 2]
    g, h, i = X[..., 2, 0], X[..., 2, 1], X[..., 2, 2]
    c00 = e * i - f * h
    c01 = f * g - d * i
    c02 = d * h - e * g
    c10 = c * h - b * i
    c11 = a * i - c * g
    c12 = b * g - a * h
    c20 = b * f - c * e
    c21 = c * d - a * f
    c22 = a * e - b * d
    det = a * c00 + b * c01 + c * c02
    r0 = jnp.stack([c00, c01, c02], axis=-1)
    r1 = jnp.stack([c10, c11, c12], axis=-1)
    r2 = jnp.stack([c20, c21, c22], axis=-1)
    return jnp.stack([r0, r1, r2], axis=-2) / det[..., None, None]


def _svd_rt(H, cs, ct):
    nf = jnp.sqrt(jnp.sum(H * H, axis=(-2, -1), keepdims=True))
    X = H / nf
    for _ in range(9):
        X = 0.5 * (X + _inv_t(X))
    R = jnp.swapaxes(X, -1, -2)
    t = ct - jnp.einsum('...ij,...j->...i', R, cs)
    return R, t


def kernel(source, target):
    B, N, _ = source.shape
    st0 = jnp.swapaxes(source, 1, 2)
    gt = jnp.swapaxes(target, 1, 2)

    tt = st0
    Rp = jnp.broadcast_to(jnp.eye(3, dtype=F32), (B, 3, 3))
    tp = jnp.zeros((B, 1, 3), F32)
    for step in range(STEPS_):
        H, cs, ct, tt = _match_call(tt, gt, Rp, tp, apply_tf=(step > 0))
        Rp, t = _svd_rt(H, cs[:, 0, :], ct[:, 0, :])
        tp = t[:, None, :]
    H, cs, ct = _final_call(st0, tt, Rp, tp)
    R, t = _svd_rt(H, cs[:, 0, :], ct[:, 0, :])
    return jnp.concatenate([R, t[..., None]], axis=-1)
